# baseline (speedup 1.0000x reference)
import functools

import jax
import jax.numpy as jnp
from jax import lax
from jax.experimental import pallas as pl
from jax.experimental.pallas import tpu as pltpu
from jax.experimental.pallas import tpu_sc as plsc

_NC = 2
_NS = 16
_NW = _NC * _NS

_B = 1024 * 200
_D = 128
_BPW = _B // _NW
_CHUNK = 128
_NCHUNK = _BPW // _CHUNK
_NPAIR = _NCHUNK // 2
_NRING = 3


def _build_gather():
    mesh = plsc.VectorSubcoreMesh(core_axis_name="c", subcore_axis_name="s")

    @functools.partial(
        pl.kernel,
        mesh=mesh,
        out_type=jax.ShapeDtypeStruct((_NW, _NCHUNK, _CHUNK, _D), jnp.float32),
        scratch_types=[
            pltpu.VMEM((_NCHUNK, _CHUNK), jnp.int32),
            pltpu.VMEM((2 * _NRING, _CHUNK, _D), jnp.float32),
        ] + [pltpu.SemaphoreType.DMA] * (3 * _NRING),
    )
    def gather_kernel(idx_hbm, table_hbm, out_hbm, idx_v, rows_v, *sems):
        sem_g = sems[:2 * _NRING]
        sem_s = sems[2 * _NRING:]
        wid = lax.axis_index("s") * _NC + lax.axis_index("c")
        pltpu.sync_copy(idx_hbm.at[wid], idx_v)

        def start_gpair(p, r):
            for j in range(2):
                b = 2 * r + j
                pltpu.make_async_copy(
                    table_hbm.at[idx_v.at[2 * p + j]], rows_v.at[b],
                    sem_g[b]).start()

        def wait_gpair(p, r):
            for j in range(2):
                b = 2 * r + j
                pltpu.make_async_copy(
                    table_hbm.at[idx_v.at[2 * p + j]], rows_v.at[b],
                    sem_g[b]).wait()

        def start_spair(p, r):
            pltpu.make_async_copy(
                rows_v.at[pl.ds(2 * r, 2)],
                out_hbm.at[wid, pl.ds(2 * p, 2)], sem_s[r]).start()

        def wait_spair(p, r):
            pltpu.make_async_copy(
                rows_v.at[pl.ds(2 * r, 2)],
                out_hbm.at[wid, pl.ds(2 * p, 2)], sem_s[r]).wait()

        def step(p, r, wait_s, start_g):
            wait_gpair(p, r)
            start_spair(p, r)
            if start_g:
                r2 = (r + 2) % _NRING
                if wait_s:
                    wait_spair(p, r2)
                start_gpair(p + 2, r2)

        start_gpair(0, 0)
        start_gpair(1, 1)
        step(0, 0, wait_s=False, start_g=True)
        step(1, 1, wait_s=True, start_g=True)
        step(2, 2, wait_s=True, start_g=True)

        n_groups = (_NPAIR - 2 - _NRING) // _NRING

        def group(g, carry):
            p0 = g * _NRING
            for r in range(_NRING):
                step(p0 + r, r, wait_s=True, start_g=True)
            return carry

        lax.fori_loop(1, 1 + n_groups, group, 0)

        for p in range(_NRING * (1 + n_groups), _NPAIR):
            step(p, p % _NRING, wait_s=True, start_g=(p + 2 < _NPAIR))
        for p in range(_NPAIR - _NRING, _NPAIR):
            wait_spair(p, p % _NRING)

    return gather_kernel


_GATHER = _build_gather()


def kernel(x, table):
    xf = x.reshape(_NW, _NCHUNK, _CHUNK).astype(jnp.int32)
    out = _GATHER(xf, table)
    return out.reshape(x.shape[0], x.shape[1], _D)


# speedup vs baseline: 1.0130x; 1.0130x over previous
import functools

import jax
import jax.numpy as jnp
from jax import lax
from jax.experimental import pallas as pl
from jax.experimental.pallas import tpu as pltpu
from jax.experimental.pallas import tpu_sc as plsc

_NC = 2
_NS = 16
_NW = _NC * _NS

_B = 1024 * 200
_D = 128
_BPW = _B // _NW
_CHUNK = 128
_NCHUNK = _BPW // _CHUNK
_NBUF = 3


def _build_gather():
    mesh = plsc.VectorSubcoreMesh(core_axis_name="c", subcore_axis_name="s")

    @functools.partial(
        pl.kernel,
        mesh=mesh,
        out_type=jax.ShapeDtypeStruct((_NW, _NCHUNK, _CHUNK, _D), jnp.float32),
        scratch_types=[
            pltpu.VMEM((_NCHUNK, _CHUNK), jnp.int32),
            pltpu.VMEM((_NBUF, _CHUNK, _D), jnp.float32),
            pltpu.VMEM_SHARED((_NS, _NBUF, _CHUNK, _D), jnp.float32),
        ] + [pltpu.SemaphoreType.DMA] * (3 * _NBUF),
    )
    def gather_kernel(idx_hbm, table_hbm, out_hbm, idx_v, rows_v, rows_sh,
                      *sems):
        sem_g = sems[:_NBUF]
        sem_c = sems[_NBUF:2 * _NBUF]
        sem_s = sems[2 * _NBUF:]
        sid = lax.axis_index("s")
        wid = sid * _NC + lax.axis_index("c")
        pltpu.sync_copy(idx_hbm.at[wid], idx_v)

        def start_g(i, b):
            pltpu.make_async_copy(
                table_hbm.at[idx_v.at[i]], rows_v.at[b], sem_g[b]).start()

        def wait_g(i, b):
            pltpu.make_async_copy(
                table_hbm.at[idx_v.at[i]], rows_v.at[b], sem_g[b]).wait()

        def start_c(b):
            pltpu.make_async_copy(
                rows_v.at[b], rows_sh.at[sid, b], sem_c[b]).start()

        def wait_c(b):
            pltpu.make_async_copy(
                rows_v.at[b], rows_sh.at[sid, b], sem_c[b]).wait()

        def start_s(i, b):
            pltpu.make_async_copy(
                rows_sh.at[sid, b], out_hbm.at[wid, i], sem_s[b]).start()

        def wait_s(i, b):
            pltpu.make_async_copy(
                rows_sh.at[sid, b], out_hbm.at[wid, i], sem_s[b]).wait()

        def step(i, b, wait_sl, do_c2, start_gather):
            wait_g(i, b)
            if wait_sl:
                wait_s(i, b)
            start_c(b)
            if do_c2:
                bj = (b - 1) % _NBUF
                wait_c(bj)
                start_s(i - 1, bj)
            if start_gather:
                start_g(i + 2, (b + 2) % _NBUF)

        start_g(0, 0)
        start_g(1, 1)
        for i in range(_NBUF):
            step(i, i, wait_sl=False, do_c2=(i >= 1), start_gather=True)

        n_groups = (_NCHUNK - 2 - _NBUF) // _NBUF

        def group(g, carry):
            i0 = g * _NBUF
            for b in range(_NBUF):
                step(i0 + b, b, wait_sl=True, do_c2=True, start_gather=True)
            return carry

        lax.fori_loop(1, 1 + n_groups, group, 0)

        for i in range(_NBUF * (1 + n_groups), _NCHUNK):
            step(i, i % _NBUF, wait_sl=True, do_c2=True,
                 start_gather=(i + 2 < _NCHUNK))
        last = _NCHUNK - 1
        wait_c(last % _NBUF)
        start_s(last, last % _NBUF)
        for i in range(_NCHUNK - _NBUF, _NCHUNK):
            wait_s(i, i % _NBUF)

    return gather_kernel


_GATHER = _build_gather()


def kernel(x, table):
    xf = x.reshape(_NW, _NCHUNK, _CHUNK).astype(jnp.int32)
    out = _GATHER(xf, table)
    return out.reshape(x.shape[0], x.shape[1], _D)
